# baseline (speedup 1.0000x reference)
import functools

import jax
import jax.numpy as jnp
from jax import lax
from jax.experimental import pallas as pl
from jax.experimental.pallas import tpu as pltpu
from jax.experimental.pallas import tpu_sc as plsc

N = 1600000
G = 4096
NCH = 4
NC = 2
NS = 16
L = 16
NW = NC * NS
CHUNK = N // NW
BLK = 10000
NB = CHUNK // BLK
VPB = BLK // L

_mesh = plsc.VectorSubcoreMesh(core_axis_name="c", subcore_axis_name="s")


def _lane_gather(x, idx):
    dnums = lax.GatherDimensionNumbers(
        offset_dims=(), collapsed_slice_dims=(0,), start_index_map=(0,))
    return lax.gather(x, idx[:, None], dnums, (1,),
                      mode=lax.GatherScatterMode.PROMISE_IN_BOUNDS)


def _cumsum16(x, iota):
    for d in (1, 2, 4, 8):
        shifted = _lane_gather(x, jnp.maximum(iota - d, 0))
        x = x + jnp.where(iota >= d, shifted, 0.0)
    return x


@functools.partial(
    pl.kernel,
    mesh=_mesh,
    out_type=jax.ShapeDtypeStruct((NW, NCH, G), jnp.float32),
    compiler_params=pltpu.CompilerParams(needs_layout_passes=False),
    scratch_types=[
        pltpu.VMEM((BLK,), jnp.int32),
        pltpu.VMEM((BLK,), jnp.float32),
        pltpu.VMEM((BLK,), jnp.float32),
        pltpu.VMEM((BLK,), jnp.float32),
        pltpu.VMEM((BLK,), jnp.float32),
        pltpu.VMEM((BLK,), jnp.float32),
        pltpu.VMEM((G,), jnp.float32),
        pltpu.VMEM((G,), jnp.float32),
        pltpu.VMEM((G,), jnp.float32),
        pltpu.VMEM((G,), jnp.float32),
        pltpu.VMEM((L,), jnp.float32),
        pltpu.VMEM((L,), jnp.float32),
        pltpu.VMEM((L,), jnp.float32),
        pltpu.VMEM((L,), jnp.float32),
        pltpu.VMEM((L,), jnp.int32),
        pltpu.VMEM((L,), jnp.int32),
    ],
)
def _sc_partials(batch_hbm, e_hbm, se_hbm, y_hbm, p0_hbm, p1_hbm, out_hbm,
                 b_buf, e_buf, se_buf, y_buf, p0_buf, p1_buf,
                 acc_a, acc_b, acc_f, acc_g,
                 vacc_a, vacc_b, vacc_f, vacc_g, prev_ref, pv_buf):
    wid = lax.axis_index("s") * NC + lax.axis_index("c")
    base = wid * CHUNK
    zero = jnp.zeros((L,), jnp.float32)
    iota = lax.iota(jnp.int32, L)
    lane0 = iota == 0
    izero = jnp.zeros((L,), jnp.int32)
    ilast = izero + (L - 1)

    def zinit(i, carry):
        off = i * L
        acc_a[pl.ds(off, L)] = zero
        acc_b[pl.ds(off, L)] = zero
        acc_f[pl.ds(off, L)] = zero
        acc_g[pl.ds(off, L)] = zero
        return carry

    lax.fori_loop(0, G // L, zinit, 0)
    vacc_a[...] = zero
    vacc_b[...] = zero
    vacc_f[...] = zero
    vacc_g[...] = zero
    pltpu.sync_copy(batch_hbm.at[pl.ds(base, L)], pv_buf)
    prev_ref[...] = _lane_gather(pv_buf[...], izero)

    def _flush(prev):
        sa = jnp.zeros((L,), jnp.float32) + jnp.sum(vacc_a[...])
        sb = jnp.zeros((L,), jnp.float32) + jnp.sum(vacc_b[...])
        sf = jnp.zeros((L,), jnp.float32) + jnp.sum(vacc_f[...])
        sg = jnp.zeros((L,), jnp.float32) + jnp.sum(vacc_g[...])
        plsc.addupdate_scatter(acc_a, [prev], sa, mask=lane0)
        plsc.addupdate_scatter(acc_b, [prev], sb, mask=lane0)
        plsc.addupdate_scatter(acc_f, [prev], sf, mask=lane0)
        plsc.addupdate_scatter(acc_g, [prev], sg, mask=lane0)

    def do_block(bi, carry):
        blk_base = base + bi * BLK
        pltpu.sync_copy(batch_hbm.at[pl.ds(blk_base, BLK)], b_buf)
        pltpu.sync_copy(e_hbm.at[pl.ds(blk_base, BLK)], e_buf)
        pltpu.sync_copy(se_hbm.at[pl.ds(blk_base, BLK)], se_buf)
        pltpu.sync_copy(y_hbm.at[pl.ds(blk_base, BLK)], y_buf)
        pltpu.sync_copy(p0_hbm.at[pl.ds(blk_base, BLK)], p0_buf)
        pltpu.sync_copy(p1_hbm.at[pl.ds(blk_base, BLK)], p1_buf)

        def do_vreg(j, vcarry):
            off = j * L
            bv = b_buf[pl.ds(off, L)]
            ev = e_buf[pl.ds(off, L)]
            sev = se_buf[pl.ds(off, L)]
            yv = y_buf[pl.ds(off, L)]
            p0 = p0_buf[pl.ds(off, L)]
            p1 = p1_buf[pl.ds(off, L)]
            emsv = ev - sev
            d0 = p0 - yv
            d1 = p1 - (1.0 - yv)
            fv = sev * d0 * d0
            gv = emsv * d1 * d1

            prev = prev_ref[...]
            bshift = _lane_gather(bv, jnp.minimum(iota + 1, L - 1))
            mb = bv != bshift
            mixed = jnp.any(mb | (bv != prev))

            @pl.when(jnp.logical_not(mixed))
            def _fast():
                vacc_a[...] = vacc_a[...] + ev
                vacc_b[...] = vacc_b[...] + emsv
                vacc_f[...] = vacc_f[...] + fv
                vacc_g[...] = vacc_g[...] + gv

            @pl.when(mixed)
            def _slow():
                _flush(prev)
                vacc_a[...] = zero
                vacc_b[...] = zero
                vacc_f[...] = zero
                vacc_g[...] = zero
                m_end = mb | (iota == L - 1)
                ca = _cumsum16(ev, iota)
                cb = _cumsum16(emsv, iota)
                cf = _cumsum16(fv, iota)
                cg = _cumsum16(gv, iota)
                plsc.addupdate_scatter(acc_a, [bv], ca, mask=m_end)
                plsc.addupdate_scatter(acc_b, [bv], cb, mask=m_end)
                plsc.addupdate_scatter(acc_f, [bv], cf, mask=m_end)
                plsc.addupdate_scatter(acc_g, [bv], cg, mask=m_end)
                plsc.addupdate_scatter(acc_a, [bshift], -ca, mask=mb)
                plsc.addupdate_scatter(acc_b, [bshift], -cb, mask=mb)
                plsc.addupdate_scatter(acc_f, [bshift], -cf, mask=mb)
                plsc.addupdate_scatter(acc_g, [bshift], -cg, mask=mb)

            prev_ref[...] = _lane_gather(bv, ilast)
            return vcarry

        return lax.fori_loop(0, VPB, do_vreg, carry)

    lax.fori_loop(0, NB, do_block, 0)
    _flush(prev_ref[...])

    pltpu.sync_copy(acc_a, out_hbm.at[wid, 0])
    pltpu.sync_copy(acc_b, out_hbm.at[wid, 1])
    pltpu.sync_copy(acc_f, out_hbm.at[wid, 2])
    pltpu.sync_copy(acc_g, out_hbm.at[wid, 3])


def _combine_body(x_ref, o_ref):
    t = jnp.sum(x_ref[...], axis=0)
    a = t[0:1, :]
    b = t[1:2, :]
    f = t[2:3, :]
    g = t[3:4, :]
    batches = f / (1.0 + a) + g / (1.0 + b)
    o_ref[...] = jnp.full((1, 1), jnp.sum(batches), jnp.float32)


_combine = pl.pallas_call(
    _combine_body,
    out_shape=jax.ShapeDtypeStruct((1, 1), jnp.float32),
)


def kernel(preds, e, shared_e, y, batch):
    batch = batch.astype(jnp.int32)
    p0 = preds[:, 0]
    p1 = preds[:, 1]
    partials = _sc_partials(batch, e, shared_e, y, p0, p1)
    return _combine(partials)[0, 0]


# speedup vs baseline: 1.3696x; 1.3696x over previous
import functools

import jax
import jax.numpy as jnp
from jax import lax
from jax.experimental import pallas as pl
from jax.experimental.pallas import tpu as pltpu
from jax.experimental.pallas import tpu_sc as plsc

N = 1600000
G = 4096
NCH = 4
NC = 2
NS = 16
L = 16
NW = NC * NS
CHUNK = N // NW
BLK = 10000
NB = CHUNK // BLK
VPB = BLK // L
K = 5
WPB = VPB // K

_mesh = plsc.VectorSubcoreMesh(core_axis_name="c", subcore_axis_name="s")


def _lane_gather(x, idx):
    dnums = lax.GatherDimensionNumbers(
        offset_dims=(), collapsed_slice_dims=(0,), start_index_map=(0,))
    return lax.gather(x, idx[:, None], dnums, (1,),
                      mode=lax.GatherScatterMode.PROMISE_IN_BOUNDS)


def _cumsum16(x, iota):
    for d in (1, 2, 4, 8):
        shifted = _lane_gather(x, jnp.maximum(iota - d, 0))
        x = x + jnp.where(iota >= d, shifted, 0.0)
    return x


@functools.partial(
    pl.kernel,
    mesh=_mesh,
    out_type=jax.ShapeDtypeStruct((NW, NCH, G), jnp.float32),
    compiler_params=pltpu.CompilerParams(needs_layout_passes=False),
    scratch_types=[
        pltpu.VMEM((BLK,), jnp.int32),
        pltpu.VMEM((BLK,), jnp.float32),
        pltpu.VMEM((BLK,), jnp.float32),
        pltpu.VMEM((BLK,), jnp.float32),
        pltpu.VMEM((BLK,), jnp.float32),
        pltpu.VMEM((BLK,), jnp.float32),
        pltpu.VMEM((G,), jnp.float32),
        pltpu.VMEM((G,), jnp.float32),
        pltpu.VMEM((G,), jnp.float32),
        pltpu.VMEM((G,), jnp.float32),
        pltpu.VMEM((L,), jnp.int32),
    ],
)
def _sc_partials(batch_hbm, e_hbm, se_hbm, y_hbm, p0_hbm, p1_hbm, out_hbm,
                 b_buf, e_buf, se_buf, y_buf, p0_buf, p1_buf,
                 acc_a, acc_b, acc_f, acc_g, pv_buf):
    wid = lax.axis_index("s") * NC + lax.axis_index("c")
    base = wid * CHUNK
    zero = jnp.zeros((L,), jnp.float32)
    iota = lax.iota(jnp.int32, L)
    lane0 = iota == 0
    izero = jnp.zeros((L,), jnp.int32)
    ilast = izero + (L - 1)
    wincheck = jnp.where(iota < 8, 0, K * L - 1)

    def zinit(i, carry):
        off = i * L
        acc_a[pl.ds(off, L)] = zero
        acc_b[pl.ds(off, L)] = zero
        acc_f[pl.ds(off, L)] = zero
        acc_g[pl.ds(off, L)] = zero
        return carry

    lax.fori_loop(0, G // L, zinit, 0)
    pltpu.sync_copy(batch_hbm.at[pl.ds(base, L)], pv_buf)
    prev0 = _lane_gather(pv_buf[...], izero)

    def _flush(prev, va, vb, vf, vg):
        plsc.addupdate_scatter(acc_a, [prev], zero + jnp.sum(va), mask=lane0)
        plsc.addupdate_scatter(acc_b, [prev], zero + jnp.sum(vb), mask=lane0)
        plsc.addupdate_scatter(acc_f, [prev], zero + jnp.sum(vf), mask=lane0)
        plsc.addupdate_scatter(acc_g, [prev], zero + jnp.sum(vg), mask=lane0)

    def _channels(off):
        ev = e_buf[pl.ds(off, L)]
        sev = se_buf[pl.ds(off, L)]
        yv = y_buf[pl.ds(off, L)]
        p0 = p0_buf[pl.ds(off, L)]
        p1 = p1_buf[pl.ds(off, L)]
        emsv = ev - sev
        d0 = p0 - yv
        d1 = p1 - (1.0 - yv)
        fv = sev * d0 * d0
        gv = emsv * d1 * d1
        return ev, emsv, fv, gv

    def do_block(bi, carry):
        blk_base = base + bi * BLK
        pltpu.sync_copy(batch_hbm.at[pl.ds(blk_base, BLK)], b_buf)
        pltpu.sync_copy(e_hbm.at[pl.ds(blk_base, BLK)], e_buf)
        pltpu.sync_copy(se_hbm.at[pl.ds(blk_base, BLK)], se_buf)
        pltpu.sync_copy(y_hbm.at[pl.ds(blk_base, BLK)], y_buf)
        pltpu.sync_copy(p0_hbm.at[pl.ds(blk_base, BLK)], p0_buf)
        pltpu.sync_copy(p1_hbm.at[pl.ds(blk_base, BLK)], p1_buf)

        def do_win(w, wcarry):
            va, vb, vf, vg, prev = wcarry
            off = w * (K * L)
            vfl = plsc.load_gather(b_buf, [off + wincheck])
            mixed = jnp.any(vfl != prev)

            def fast(ops):
                va, vb, vf, vg, prev = ops
                for k in range(K):
                    ev, emsv, fv, gv = _channels(off + k * L)
                    va = va + ev
                    vb = vb + emsv
                    vf = vf + fv
                    vg = vg + gv
                return va, vb, vf, vg, prev

            def slow(ops):
                va, vb, vf, vg, prev = ops
                _flush(prev, va, vb, vf, vg)
                for k in range(K):
                    offk = off + k * L
                    bv = b_buf[pl.ds(offk, L)]
                    ev, emsv, fv, gv = _channels(offk)
                    bshift = _lane_gather(bv, jnp.minimum(iota + 1, L - 1))
                    mb = bv != bshift
                    m_end = mb | (iota == L - 1)
                    ca = _cumsum16(ev, iota)
                    cb = _cumsum16(emsv, iota)
                    cf = _cumsum16(fv, iota)
                    cg = _cumsum16(gv, iota)
                    plsc.addupdate_scatter(acc_a, [bv], ca, mask=m_end)
                    plsc.addupdate_scatter(acc_b, [bv], cb, mask=m_end)
                    plsc.addupdate_scatter(acc_f, [bv], cf, mask=m_end)
                    plsc.addupdate_scatter(acc_g, [bv], cg, mask=m_end)
                    plsc.addupdate_scatter(acc_a, [bshift], -ca, mask=mb)
                    plsc.addupdate_scatter(acc_b, [bshift], -cb, mask=mb)
                    plsc.addupdate_scatter(acc_f, [bshift], -cf, mask=mb)
                    plsc.addupdate_scatter(acc_g, [bshift], -cg, mask=mb)
                newprev = _lane_gather(bv, ilast)
                return zero, zero, zero, zero, newprev

            return lax.cond(mixed, slow, fast, (va, vb, vf, vg, prev))

        return lax.fori_loop(0, WPB, do_win, carry)

    va, vb, vf, vg, prev = lax.fori_loop(
        0, NB, do_block, (zero, zero, zero, zero, prev0))
    _flush(prev, va, vb, vf, vg)

    pltpu.sync_copy(acc_a, out_hbm.at[wid, 0])
    pltpu.sync_copy(acc_b, out_hbm.at[wid, 1])
    pltpu.sync_copy(acc_f, out_hbm.at[wid, 2])
    pltpu.sync_copy(acc_g, out_hbm.at[wid, 3])


def _combine_body(x_ref, o_ref):
    t = jnp.sum(x_ref[...], axis=0)
    a = t[0:1, :]
    b = t[1:2, :]
    f = t[2:3, :]
    g = t[3:4, :]
    batches = f / (1.0 + a) + g / (1.0 + b)
    o_ref[...] = jnp.full((1, 1), jnp.sum(batches), jnp.float32)


_combine = pl.pallas_call(
    _combine_body,
    out_shape=jax.ShapeDtypeStruct((1, 1), jnp.float32),
)


def kernel(preds, e, shared_e, y, batch):
    batch = batch.astype(jnp.int32)
    p0 = preds[:, 0]
    p1 = preds[:, 1]
    partials = _sc_partials(batch, e, shared_e, y, p0, p1)
    return _combine(partials)[0, 0]


# speedup vs baseline: 1.6447x; 1.2008x over previous
import functools

import jax
import jax.numpy as jnp
from jax import lax
from jax.experimental import pallas as pl
from jax.experimental.pallas import tpu as pltpu
from jax.experimental.pallas import tpu_sc as plsc

N = 1600000
G = 4096
NCH = 4
NC = 2
NS = 16
L = 16
NW = NC * NS
CHUNK = N // NW
BLK = 2000
NB = CHUNK // BLK
VPB = BLK // L
K = 5
WPB = VPB // K

_mesh = plsc.VectorSubcoreMesh(core_axis_name="c", subcore_axis_name="s")


def _lane_gather(x, idx):
    dnums = lax.GatherDimensionNumbers(
        offset_dims=(), collapsed_slice_dims=(0,), start_index_map=(0,))
    return lax.gather(x, idx[:, None], dnums, (1,),
                      mode=lax.GatherScatterMode.PROMISE_IN_BOUNDS)


def _cumsum16(x, iota):
    for d in (1, 2, 4, 8):
        shifted = _lane_gather(x, jnp.maximum(iota - d, 0))
        x = x + jnp.where(iota >= d, shifted, 0.0)
    return x


@functools.partial(
    pl.kernel,
    mesh=_mesh,
    out_type=jax.ShapeDtypeStruct((NW, NCH, G), jnp.float32),
    compiler_params=pltpu.CompilerParams(needs_layout_passes=False),
    scratch_types=(
        [pltpu.VMEM((BLK,), jnp.int32) for _ in range(2)]
        + [pltpu.VMEM((BLK,), jnp.float32) for _ in range(10)]
        + [
            pltpu.VMEM((G,), jnp.float32),
            pltpu.VMEM((G,), jnp.float32),
            pltpu.VMEM((G,), jnp.float32),
            pltpu.VMEM((G,), jnp.float32),
            pltpu.VMEM((L,), jnp.int32),
            pltpu.SemaphoreType.DMA,
            pltpu.SemaphoreType.DMA,
        ]
    ),
)
def _sc_partials(batch_hbm, e_hbm, se_hbm, y_hbm, p0_hbm, p1_hbm, out_hbm,
                 b_buf0, b_buf1, e_buf0, e_buf1, se_buf0, se_buf1,
                 y_buf0, y_buf1, p0_buf0, p0_buf1, p1_buf0, p1_buf1,
                 acc_a, acc_b, acc_f, acc_g, pv_buf, sem0, sem1):
    wid = lax.axis_index("s") * NC + lax.axis_index("c")
    base = wid * CHUNK
    zero = jnp.zeros((L,), jnp.float32)
    iota = lax.iota(jnp.int32, L)
    lane0 = iota == 0
    izero = jnp.zeros((L,), jnp.int32)
    ilast = izero + (L - 1)
    wincheck = jnp.where(iota < 8, 0, K * L - 1)

    def zinit(i, carry):
        off = i * L
        acc_a[pl.ds(off, L)] = zero
        acc_b[pl.ds(off, L)] = zero
        acc_f[pl.ds(off, L)] = zero
        acc_g[pl.ds(off, L)] = zero
        return carry

    lax.fori_loop(0, G // L, zinit, 0)
    pltpu.sync_copy(batch_hbm.at[pl.ds(base, L)], pv_buf)
    prev0 = _lane_gather(pv_buf[...], izero)

    def _flush(prev, va, vb, vf, vg):
        plsc.addupdate_scatter(acc_a, [prev], zero + jnp.sum(va), mask=lane0)
        plsc.addupdate_scatter(acc_b, [prev], zero + jnp.sum(vb), mask=lane0)
        plsc.addupdate_scatter(acc_f, [prev], zero + jnp.sum(vf), mask=lane0)
        plsc.addupdate_scatter(acc_g, [prev], zero + jnp.sum(vg), mask=lane0)

    sets = (
        (b_buf0, e_buf0, se_buf0, y_buf0, p0_buf0, p1_buf0, sem0),
        (b_buf1, e_buf1, se_buf1, y_buf1, p0_buf1, p1_buf1, sem1),
    )

    def _hbm_slices(bi):
        blk_base = base + bi * BLK
        sl = pl.ds(blk_base, BLK)
        return (batch_hbm.at[sl], e_hbm.at[sl], se_hbm.at[sl],
                y_hbm.at[sl], p0_hbm.at[sl], p1_hbm.at[sl])

    def _issue(bi, bset):
        *bufs, sem = bset
        for src, dst in zip(_hbm_slices(bi), bufs):
            pltpu.async_copy(src, dst, sem)

    def _drain(bi, bset):
        *bufs, sem = bset
        for src, dst in zip(_hbm_slices(bi), bufs):
            pltpu.make_async_copy(src, dst, sem).wait()

    def _make_channels(e_buf, se_buf, y_buf, p0_buf, p1_buf):
        def _channels(off):
            ev = e_buf[pl.ds(off, L)]
            sev = se_buf[pl.ds(off, L)]
            yv = y_buf[pl.ds(off, L)]
            p0 = p0_buf[pl.ds(off, L)]
            p1 = p1_buf[pl.ds(off, L)]
            emsv = ev - sev
            d0 = p0 - yv
            d1 = p1 - (1.0 - yv)
            fv = sev * d0 * d0
            gv = emsv * d1 * d1
            return ev, emsv, fv, gv
        return _channels

    def _process(bset, carry):
        b_buf, e_buf, se_buf, y_buf, p0_buf, p1_buf, _ = bset
        _channels = _make_channels(e_buf, se_buf, y_buf, p0_buf, p1_buf)

        def do_win(w, wcarry):
            va, vb, vf, vg, prev = wcarry
            off = w * (K * L)
            vfl = plsc.load_gather(b_buf, [off + wincheck])
            mixed = jnp.any(vfl != prev)

            def fast(ops):
                va, vb, vf, vg, prev = ops
                for k in range(K):
                    ev, emsv, fv, gv = _channels(off + k * L)
                    va = va + ev
                    vb = vb + emsv
                    vf = vf + fv
                    vg = vg + gv
                return va, vb, vf, vg, prev

            def slow(ops):
                va, vb, vf, vg, prev = ops
                _flush(prev, va, vb, vf, vg)
                for k in range(K):
                    offk = off + k * L
                    bv = b_buf[pl.ds(offk, L)]
                    ev, emsv, fv, gv = _channels(offk)
                    bshift = _lane_gather(bv, jnp.minimum(iota + 1, L - 1))
                    mb = bv != bshift
                    m_end = mb | (iota == L - 1)
                    ca = _cumsum16(ev, iota)
                    cb = _cumsum16(emsv, iota)
                    cf = _cumsum16(fv, iota)
                    cg = _cumsum16(gv, iota)
                    plsc.addupdate_scatter(acc_a, [bv], ca, mask=m_end)
                    plsc.addupdate_scatter(acc_b, [bv], cb, mask=m_end)
                    plsc.addupdate_scatter(acc_f, [bv], cf, mask=m_end)
                    plsc.addupdate_scatter(acc_g, [bv], cg, mask=m_end)
                    plsc.addupdate_scatter(acc_a, [bshift], -ca, mask=mb)
                    plsc.addupdate_scatter(acc_b, [bshift], -cb, mask=mb)
                    plsc.addupdate_scatter(acc_f, [bshift], -cf, mask=mb)
                    plsc.addupdate_scatter(acc_g, [bshift], -cg, mask=mb)
                newprev = _lane_gather(bv, ilast)
                return zero, zero, zero, zero, newprev

            return lax.cond(mixed, slow, fast, (va, vb, vf, vg, prev))

        return lax.fori_loop(0, WPB, do_win, carry)

    _issue(0, sets[0])
    _issue(1, sets[1])

    def do_pair(t, carry):
        b0 = 2 * t
        _drain(b0, sets[0])
        carry = _process(sets[0], carry)
        _issue(b0 + 2, sets[0])
        _drain(b0 + 1, sets[1])
        carry = _process(sets[1], carry)

        @pl.when(b0 + 3 < NB)
        def _():
            _issue(b0 + 3, sets[1])

        return carry

    carry = lax.fori_loop(0, (NB - 1) // 2, do_pair,
                          (zero, zero, zero, zero, prev0))
    _drain(NB - 1, sets[0])
    va, vb, vf, vg, prev = _process(sets[0], carry)
    _flush(prev, va, vb, vf, vg)

    pltpu.sync_copy(acc_a, out_hbm.at[wid, 0])
    pltpu.sync_copy(acc_b, out_hbm.at[wid, 1])
    pltpu.sync_copy(acc_f, out_hbm.at[wid, 2])
    pltpu.sync_copy(acc_g, out_hbm.at[wid, 3])


def _combine_body(x_ref, o_ref):
    t = jnp.sum(x_ref[...], axis=0)
    a = t[0:1, :]
    b = t[1:2, :]
    f = t[2:3, :]
    g = t[3:4, :]
    batches = f / (1.0 + a) + g / (1.0 + b)
    o_ref[...] = jnp.full((1, 1), jnp.sum(batches), jnp.float32)


_combine = pl.pallas_call(
    _combine_body,
    out_shape=jax.ShapeDtypeStruct((1, 1), jnp.float32),
)


def kernel(preds, e, shared_e, y, batch):
    batch = batch.astype(jnp.int32)
    p0 = preds[:, 0]
    p1 = preds[:, 1]
    partials = _sc_partials(batch, e, shared_e, y, p0, p1)
    return _combine(partials)[0, 0]


# speedup vs baseline: 1.6774x; 1.0199x over previous
import functools

import jax
import jax.numpy as jnp
from jax import lax
from jax.experimental import pallas as pl
from jax.experimental.pallas import tpu as pltpu
from jax.experimental.pallas import tpu_sc as plsc

N = 1600000
G = 4096
NCH = 4
NC = 2
NS = 16
L = 16
NW = NC * NS
CHUNK = N // NW
BLK = 2000
NB = CHUNK // BLK
VPB = BLK // L
K = 5
WPB = VPB // K

_mesh = plsc.VectorSubcoreMesh(core_axis_name="c", subcore_axis_name="s")


def _lane_gather(x, idx):
    dnums = lax.GatherDimensionNumbers(
        offset_dims=(), collapsed_slice_dims=(0,), start_index_map=(0,))
    return lax.gather(x, idx[:, None], dnums, (1,),
                      mode=lax.GatherScatterMode.PROMISE_IN_BOUNDS)


def _cumsum16(x, iota):
    del iota
    return plsc.cumsum(x)


@functools.partial(
    pl.kernel,
    mesh=_mesh,
    out_type=jax.ShapeDtypeStruct((NW, NCH, G), jnp.float32),
    compiler_params=pltpu.CompilerParams(needs_layout_passes=False),
    scratch_types=(
        [pltpu.VMEM((BLK,), jnp.int32) for _ in range(2)]
        + [pltpu.VMEM((BLK,), jnp.float32) for _ in range(10)]
        + [
            pltpu.VMEM((G,), jnp.float32),
            pltpu.VMEM((G,), jnp.float32),
            pltpu.VMEM((G,), jnp.float32),
            pltpu.VMEM((G,), jnp.float32),
            pltpu.VMEM((L,), jnp.int32),
            pltpu.SemaphoreType.DMA,
            pltpu.SemaphoreType.DMA,
        ]
    ),
)
def _sc_partials(batch_hbm, e_hbm, se_hbm, y_hbm, p0_hbm, p1_hbm, out_hbm,
                 b_buf0, b_buf1, e_buf0, e_buf1, se_buf0, se_buf1,
                 y_buf0, y_buf1, p0_buf0, p0_buf1, p1_buf0, p1_buf1,
                 acc_a, acc_b, acc_f, acc_g, pv_buf, sem0, sem1):
    wid = lax.axis_index("s") * NC + lax.axis_index("c")
    base = wid * CHUNK
    zero = jnp.zeros((L,), jnp.float32)
    iota = lax.iota(jnp.int32, L)
    lane0 = iota == 0
    izero = jnp.zeros((L,), jnp.int32)
    ilast = izero + (L - 1)
    wincheck = jnp.where(iota < 8, 0, K * L - 1)

    def zinit(i, carry):
        off = i * L
        acc_a[pl.ds(off, L)] = zero
        acc_b[pl.ds(off, L)] = zero
        acc_f[pl.ds(off, L)] = zero
        acc_g[pl.ds(off, L)] = zero
        return carry

    lax.fori_loop(0, G // L, zinit, 0)
    pltpu.sync_copy(batch_hbm.at[pl.ds(base, L)], pv_buf)
    prev0 = _lane_gather(pv_buf[...], izero)

    def _flush(prev, va, vb, vf, vg):
        plsc.addupdate_scatter(acc_a, [prev], zero + jnp.sum(va), mask=lane0)
        plsc.addupdate_scatter(acc_b, [prev], zero + jnp.sum(vb), mask=lane0)
        plsc.addupdate_scatter(acc_f, [prev], zero + jnp.sum(vf), mask=lane0)
        plsc.addupdate_scatter(acc_g, [prev], zero + jnp.sum(vg), mask=lane0)

    sets = (
        (b_buf0, e_buf0, se_buf0, y_buf0, p0_buf0, p1_buf0, sem0),
        (b_buf1, e_buf1, se_buf1, y_buf1, p0_buf1, p1_buf1, sem1),
    )

    def _hbm_slices(bi):
        blk_base = base + bi * BLK
        sl = pl.ds(blk_base, BLK)
        return (batch_hbm.at[sl], e_hbm.at[sl], se_hbm.at[sl],
                y_hbm.at[sl], p0_hbm.at[sl], p1_hbm.at[sl])

    def _issue(bi, bset):
        *bufs, sem = bset
        for src, dst in zip(_hbm_slices(bi), bufs):
            pltpu.async_copy(src, dst, sem)

    def _drain(bi, bset):
        *bufs, sem = bset
        for src, dst in zip(_hbm_slices(bi), bufs):
            pltpu.make_async_copy(src, dst, sem).wait()

    def _make_channels(e_buf, se_buf, y_buf, p0_buf, p1_buf):
        def _channels(off):
            ev = e_buf[pl.ds(off, L)]
            sev = se_buf[pl.ds(off, L)]
            yv = y_buf[pl.ds(off, L)]
            p0 = p0_buf[pl.ds(off, L)]
            p1 = p1_buf[pl.ds(off, L)]
            emsv = ev - sev
            d0 = p0 - yv
            d1 = p1 - (1.0 - yv)
            fv = sev * d0 * d0
            gv = emsv * d1 * d1
            return ev, emsv, fv, gv
        return _channels

    def _process(bset, carry):
        b_buf, e_buf, se_buf, y_buf, p0_buf, p1_buf, _ = bset
        _channels = _make_channels(e_buf, se_buf, y_buf, p0_buf, p1_buf)

        def do_win(w, wcarry):
            va, vb, vf, vg, prev = wcarry
            off = w * (K * L)
            vfl = plsc.load_gather(b_buf, [off + wincheck])
            mixed = jnp.any(vfl != prev)

            def fast(ops):
                va, vb, vf, vg, prev = ops
                for k in range(K):
                    ev, emsv, fv, gv = _channels(off + k * L)
                    va = va + ev
                    vb = vb + emsv
                    vf = vf + fv
                    vg = vg + gv
                return va, vb, vf, vg, prev

            def slow(ops):
                va, vb, vf, vg, prev = ops
                _flush(prev, va, vb, vf, vg)
                for k in range(K):
                    offk = off + k * L
                    bv = b_buf[pl.ds(offk, L)]
                    ev, emsv, fv, gv = _channels(offk)
                    bshift = _lane_gather(bv, jnp.minimum(iota + 1, L - 1))
                    mb = bv != bshift
                    m_end = mb | (iota == L - 1)
                    ca = _cumsum16(ev, iota)
                    cb = _cumsum16(emsv, iota)
                    cf = _cumsum16(fv, iota)
                    cg = _cumsum16(gv, iota)
                    plsc.addupdate_scatter(acc_a, [bv], ca, mask=m_end)
                    plsc.addupdate_scatter(acc_b, [bv], cb, mask=m_end)
                    plsc.addupdate_scatter(acc_f, [bv], cf, mask=m_end)
                    plsc.addupdate_scatter(acc_g, [bv], cg, mask=m_end)
                    plsc.addupdate_scatter(acc_a, [bshift], -ca, mask=mb)
                    plsc.addupdate_scatter(acc_b, [bshift], -cb, mask=mb)
                    plsc.addupdate_scatter(acc_f, [bshift], -cf, mask=mb)
                    plsc.addupdate_scatter(acc_g, [bshift], -cg, mask=mb)
                newprev = _lane_gather(bv, ilast)
                return zero, zero, zero, zero, newprev

            return lax.cond(mixed, slow, fast, (va, vb, vf, vg, prev))

        return lax.fori_loop(0, WPB, do_win, carry)

    _issue(0, sets[0])
    _issue(1, sets[1])

    def do_pair(t, carry):
        b0 = 2 * t
        _drain(b0, sets[0])
        carry = _process(sets[0], carry)
        _issue(b0 + 2, sets[0])
        _drain(b0 + 1, sets[1])
        carry = _process(sets[1], carry)

        @pl.when(b0 + 3 < NB)
        def _():
            _issue(b0 + 3, sets[1])

        return carry

    carry = lax.fori_loop(0, (NB - 1) // 2, do_pair,
                          (zero, zero, zero, zero, prev0))
    _drain(NB - 1, sets[0])
    va, vb, vf, vg, prev = _process(sets[0], carry)
    _flush(prev, va, vb, vf, vg)

    pltpu.sync_copy(acc_a, out_hbm.at[wid, 0])
    pltpu.sync_copy(acc_b, out_hbm.at[wid, 1])
    pltpu.sync_copy(acc_f, out_hbm.at[wid, 2])
    pltpu.sync_copy(acc_g, out_hbm.at[wid, 3])


def _combine_body(x_ref, o_ref):
    t = jnp.sum(x_ref[...], axis=0)
    a = t[0:1, :]
    b = t[1:2, :]
    f = t[2:3, :]
    g = t[3:4, :]
    batches = f / (1.0 + a) + g / (1.0 + b)
    o_ref[...] = jnp.full((1, 1), jnp.sum(batches), jnp.float32)


_combine = pl.pallas_call(
    _combine_body,
    out_shape=jax.ShapeDtypeStruct((1, 1), jnp.float32),
)


def kernel(preds, e, shared_e, y, batch):
    batch = batch.astype(jnp.int32)
    p0 = preds[:, 0]
    p1 = preds[:, 1]
    partials = _sc_partials(batch, e, shared_e, y, p0, p1)
    return _combine(partials)[0, 0]
